# Initial kernel scaffold; baseline (speedup 1.0000x reference)
import jax
import jax.numpy as jnp
from jax.experimental import pallas as pl


def kernel(x_node_ids, edge_index, emb_table, W1, b1, W2, b2):
    raise NotImplementedError("write your pallas kernel here")


# speedup vs baseline: 11.2415x; 11.2415x over previous
import functools

import jax
import jax.numpy as jnp
from jax import lax
from jax.experimental import pallas as pl
from jax.experimental.pallas import tpu as pltpu
from jax.experimental.pallas import tpu_sc as plsc

N_NODES = 10000
N_EDGES = 160000
VOCAB = 512
EMB_DIM = 256
HIDDEN = 512
OUT_DIM = 256

NPAD = 10240
SLICE = NPAD // 16
RROWS = 160
NRANGE = NPAD // RROWS
ECH = 2048
EPT = N_EDGES // 16
L = 16


def _rsqrt16(d):
    i = plsc.bitcast(d, jnp.int32)
    i = jnp.int32(0x5F3759DF) - lax.shift_right_logical(i, 1)
    y = plsc.bitcast(i, jnp.float32)
    for _ in range(3):
        y = y * (1.5 - 0.5 * d * y * y)
    return y


def _sc_body(ids_hbm, src_hbm, dst_hbm, g_hbm, c_hbm,
             ids_v, dinv_v, hist_v, tmp_v, g_v, es_v, ed_v,
             part_sh, dinv_sh):
    cid = lax.axis_index("c")
    sid = lax.axis_index("s")
    wid = cid * 16 + sid
    ones = jnp.full((L,), 1.0, jnp.float32)
    zeros = jnp.zeros((L,), jnp.float32)

    pltpu.sync_copy(ids_hbm, ids_v)

    def zero_hist(j, _):
        hist_v[pl.ds(j * L, L)] = zeros
        return 0

    lax.fori_loop(0, NPAD // L, zero_hist, 0)
    ebase = sid * EPT
    slice_chunks = [(i * ECH, ECH) for i in range(EPT // ECH)]
    if EPT % ECH:
        slice_chunks.append(((EPT // ECH) * ECH, EPT % ECH))
    for off, n in slice_chunks:
        pltpu.sync_copy(dst_hbm.at[pl.ds(ebase + off, n)], ed_v.at[pl.ds(0, n)])

        def deg_body(i, _):
            d = ed_v[pl.ds(i * L, L)]
            plsc.addupdate_scatter(hist_v, [d], ones)
            return 0

        lax.fori_loop(0, n // L, deg_body, 0)

    pltpu.sync_copy(hist_v, part_sh.at[sid])
    plsc.subcore_barrier()
    cs = sid * SLICE

    def zero_acc(j, _):
        dinv_v[pl.ds(cs + j * L, L)] = zeros
        return 0

    lax.fori_loop(0, SLICE // L, zero_acc, 0)
    for k in range(16):
        pltpu.sync_copy(part_sh.at[k, pl.ds(cs, SLICE)], tmp_v)

        def add_k(j, _):
            dinv_v[pl.ds(cs + j * L, L)] += tmp_v[pl.ds(j * L, L)]
            return 0

        lax.fori_loop(0, SLICE // L, add_k, 0)

    def newton(j, _):
        deg = dinv_v[pl.ds(cs + j * L, L)] + 1.0
        dinv_v[pl.ds(cs + j * L, L)] = _rsqrt16(deg)
        return 0

    lax.fori_loop(0, SLICE // L, newton, 0)
    pltpu.sync_copy(dinv_v.at[pl.ds(cs, SLICE)], dinv_sh.at[pl.ds(cs, SLICE)])
    plsc.subcore_barrier()
    pltpu.sync_copy(dinv_sh, dinv_v)

    lax.fori_loop(0, NPAD // L, zero_hist, 0)
    for off, n in slice_chunks:
        pltpu.sync_copy(src_hbm.at[pl.ds(ebase + off, n)], es_v.at[pl.ds(0, n)])
        pltpu.sync_copy(dst_hbm.at[pl.ds(ebase + off, n)], ed_v.at[pl.ds(0, n)])

        def s_body(i, _):
            s = es_v[pl.ds(i * L, L)]
            d = ed_v[pl.ds(i * L, L)]
            plsc.addupdate_scatter(hist_v, [s], plsc.load_gather(dinv_v, [d]))
            return 0

        lax.fori_loop(0, n // L, s_body, 0)

    pltpu.sync_copy(hist_v, part_sh.at[sid])
    plsc.subcore_barrier()

    def zero_acc_h(j, _):
        hist_v[pl.ds(cs + j * L, L)] = zeros
        return 0

    lax.fori_loop(0, SLICE // L, zero_acc_h, 0)
    for k in range(16):
        pltpu.sync_copy(part_sh.at[k, pl.ds(cs, SLICE)], tmp_v)

        def add_k2(j, _):
            hist_v[pl.ds(cs + j * L, L)] += tmp_v[pl.ds(j * L, L)]
            return 0

        lax.fori_loop(0, SLICE // L, add_k2, 0)

    @pl.when(cid == 0)
    def _():
        def c_body(j, _):
            idx = lax.iota(jnp.int32, L) + (cs + j * L)
            dv = dinv_v[pl.ds(cs + j * L, L)]
            sv = hist_v[pl.ds(cs + j * L, L)]
            tmp_v[pl.ds(j * L, L)] = jnp.where(idx < N_NODES, dv * (dv + sv), 0.0)
            return 0

        lax.fori_loop(0, SLICE // L, c_body, 0)
        pltpu.sync_copy(tmp_v, c_hbm.at[pl.ds(cs, SLICE)])

    for rr in range(2):
        rng = wid * 2 + rr
        base = rng * RROWS

        def zero_g(j, _):
            g_v[pl.ds(j * L, L)] = zeros
            return 0

        lax.fori_loop(0, RROWS * VOCAB // L, zero_g, 0)

        def chunk(ch, _):
            off = ch * ECH
            pltpu.sync_copy(src_hbm.at[pl.ds(off, ECH)], es_v)
            pltpu.sync_copy(dst_hbm.at[pl.ds(off, ECH)], ed_v)

            def edge_body(i, _):
                s = es_v[pl.ds(i * L, L)]
                d = ed_v[pl.ds(i * L, L)]
                vid = plsc.load_gather(ids_v, [s])
                w = plsc.load_gather(dinv_v, [s]) * plsc.load_gather(dinv_v, [d])
                local = d - base
                msk = (local >= 0) & (local < RROWS)
                lc = jnp.where(msk, local, 0)
                plsc.addupdate_scatter(g_v, [lc * VOCAB + vid], w, mask=msk)
                return 0

            lax.fori_loop(0, ECH // L, edge_body, 0)
            return 0

        lax.fori_loop(0, N_EDGES // ECH, chunk, 0)
        tail = N_EDGES - (N_EDGES // ECH) * ECH
        if tail:
            toff = (N_EDGES // ECH) * ECH
            pltpu.sync_copy(src_hbm.at[pl.ds(toff, tail)], es_v.at[pl.ds(0, tail)])
            pltpu.sync_copy(dst_hbm.at[pl.ds(toff, tail)], ed_v.at[pl.ds(0, tail)])

            def tail_body(i, _):
                s = es_v[pl.ds(i * L, L)]
                d = ed_v[pl.ds(i * L, L)]
                vid = plsc.load_gather(ids_v, [s])
                w = plsc.load_gather(dinv_v, [s]) * plsc.load_gather(dinv_v, [d])
                local = d - base
                msk = (local >= 0) & (local < RROWS)
                lc = jnp.where(msk, local, 0)
                plsc.addupdate_scatter(g_v, [lc * VOCAB + vid], w, mask=msk)
                return 0

            lax.fori_loop(0, tail // L, tail_body, 0)

        def self_body(j, _):
            li = lax.iota(jnp.int32, L) + j * L
            node = li + base
            m = node < N_NODES
            nc = jnp.where(m, node, 0)
            vidn = plsc.load_gather(ids_v, [nc])
            dv = plsc.load_gather(dinv_v, [nc])
            plsc.addupdate_scatter(g_v, [li * VOCAB + vidn], dv * dv, mask=m)
            return 0

        lax.fori_loop(0, RROWS // L, self_body, 0)
        pltpu.sync_copy(g_v, g_hbm.at[pl.ds(base * VOCAB, RROWS * VOCAB)])


def _make_sc_call(interpret=False):
    mesh = plsc.VectorSubcoreMesh(core_axis_name="c", subcore_axis_name="s",
                                  num_cores=2, num_subcores=16)
    return pl.kernel(
        _sc_body,
        out_type=[
            jax.ShapeDtypeStruct((NPAD * VOCAB,), jnp.float32),
            jax.ShapeDtypeStruct((NPAD,), jnp.float32),
        ],
        mesh=mesh,
        scratch_types=[
            pltpu.VMEM((N_NODES,), jnp.int32),
            pltpu.VMEM((NPAD,), jnp.float32),
            pltpu.VMEM((NPAD,), jnp.float32),
            pltpu.VMEM((SLICE,), jnp.float32),
            pltpu.VMEM((RROWS * VOCAB,), jnp.float32),
            pltpu.VMEM((ECH,), jnp.int32),
            pltpu.VMEM((ECH,), jnp.int32),
            pltpu.VMEM_SHARED((16, NPAD), jnp.float32),
            pltpu.VMEM_SHARED((NPAD,), jnp.float32),
        ],
        compiler_params=pltpu.CompilerParams(needs_layout_passes=False),
        interpret=interpret,
    )


def _tc_body(g_ref, c_ref, e_ref, w1_ref, b1_ref, w2_ref, b2_ref, out_ref,
             t_s, p_s):
    i = pl.program_id(0)

    @pl.when(i == 0)
    def _():
        t_s[...] = jnp.dot(e_ref[...], w1_ref[...],
                           preferred_element_type=jnp.float32)
        p_s[...] = jnp.zeros((1, HIDDEN), jnp.float32)

    z = jnp.dot(g_ref[...], t_s[...], preferred_element_type=jnp.float32)
    h = jnp.maximum(z + b1_ref[...], 0.0)
    p_s[...] += jnp.sum(h * c_ref[0], axis=0, keepdims=True)

    @pl.when(i == NRANGE - 1)
    def _():
        out_ref[...] = jnp.dot(p_s[...] * (1.0 / N_NODES), w2_ref[...],
                               preferred_element_type=jnp.float32) + b2_ref[...]


def _make_tc_call(interpret=False):
    return pl.pallas_call(
        _tc_body,
        grid=(NRANGE,),
        in_specs=[
            pl.BlockSpec((RROWS, VOCAB), lambda i: (i, 0)),
            pl.BlockSpec((1, RROWS, 1), lambda i: (i, 0, 0)),
            pl.BlockSpec((VOCAB, EMB_DIM), lambda i: (0, 0)),
            pl.BlockSpec((EMB_DIM, HIDDEN), lambda i: (0, 0)),
            pl.BlockSpec((1, HIDDEN), lambda i: (0, 0)),
            pl.BlockSpec((HIDDEN, OUT_DIM), lambda i: (0, 0)),
            pl.BlockSpec((1, OUT_DIM), lambda i: (0, 0)),
        ],
        out_specs=pl.BlockSpec((1, OUT_DIM), lambda i: (0, 0)),
        out_shape=jax.ShapeDtypeStruct((1, OUT_DIM), jnp.float32),
        scratch_shapes=[
            pltpu.VMEM((VOCAB, HIDDEN), jnp.float32),
            pltpu.VMEM((1, HIDDEN), jnp.float32),
        ],
        interpret=interpret,
    )


@jax.jit
def kernel(x_node_ids, edge_index, emb_table, W1, b1, W2, b2):
    ids = x_node_ids.astype(jnp.int32)
    src = edge_index[0].astype(jnp.int32)
    dst = edge_index[1].astype(jnp.int32)
    g_flat, c = _make_sc_call()(ids, src, dst)
    G = g_flat.reshape(NPAD, VOCAB)
    c3 = c.reshape(NRANGE, RROWS, 1)
    return _make_tc_call()(
        G, c3, emb_table, W1, b1.reshape(1, HIDDEN), W2,
        b2.reshape(1, OUT_DIM))
